# baseline (speedup 1.0000x reference)
import functools

import jax
import jax.numpy as jnp
from jax import lax
from jax.experimental import pallas as pl
from jax.experimental.pallas import tpu as pltpu
from jax.experimental.pallas import tpu_sc as plsc

N = 10000
E = 320000
DIM = 128
G = 128
L = 3

NC = 2
NS = 16
CH = 40
E_TILE = E // (NC * NS)
STEPS = E_TILE // CH
NB = 4
NE = 8
ROWS_A = 624
ROWS_TAIL = N - NS * ROWS_A

UNROLL = 8
HEAD = 8
LOOP_LO = 1
LOOP_HI = STEPS // UNROLL
TAIL_LO = UNROLL * LOOP_HI


def _agg_body(h_hbm, eidx_hbm, zero_hbm, out_hbm, shared, *bufs):
    rows = bufs[0:NB]
    ev = bufs[NB:NB + NE]
    g = bufs[NB + NE:2 * NB + NE]
    sc = bufs[2 * NB + NE:3 * NB + NE]
    a = bufs[3 * NB + NE:3 * NB + 2 * NE]

    c = lax.axis_index("c")
    s = lax.axis_index("s")
    wid = c * NS + s

    r0 = pl.multiple_of(s * ROWS_A, 8)
    pltpu.sync_copy(zero_hbm.at[pl.ds(r0, ROWS_A)], shared.at[pl.ds(r0, ROWS_A)])

    @pl.when(s == NS - 1)
    def _ztail():
        t0 = NS * ROWS_A
        pltpu.sync_copy(zero_hbm.at[pl.ds(t0, ROWS_TAIL)],
                        shared.at[pl.ds(t0, ROWS_TAIL)])

    def _wait_rows(buf, sem):
        pltpu.make_async_copy(h_hbm.at[pl.ds(0, CH)], buf, sem).wait()

    def _wait_idx(buf, sem):
        pltpu.make_async_copy(eidx_hbm.at[wid, 0], buf, sem).wait()

    def _wait_scat(sem):
        pltpu.make_async_copy(h_hbm.at[pl.ds(0, CH)], rows[0], sem).wait()

    for k in range(3):
        pltpu.sync_copy(eidx_hbm.at[wid, k], ev[k])
    plsc.subcore_barrier()
    for k in range(3):
        pltpu.async_copy(h_hbm.at[ev[k].at[0]], rows[k], g[k])
    for k in range(3, 7):
        pltpu.async_copy(eidx_hbm.at[wid, k], ev[k], a[k])

    def _step(i, k4, k8, static):
        kn4 = (k4 + 3) % NB
        kn8 = (k8 + 3) % NE
        _wait_rows(rows[k4], g[k4])

        def _gn():
            _wait_idx(ev[kn8], a[kn8])
            pltpu.async_copy(h_hbm.at[ev[kn8].at[0]], rows[kn4], g[kn4])

        if static:
            if i + 3 < STEPS:

                _gn()
        else:
            @pl.when(i + 3 < STEPS)
            def _():
                _gn()


        if static:
            if i + 7 < STEPS:
                pltpu.async_copy(eidx_hbm.at[wid, i + 7], ev[kn8 + 4 - NE], a[kn8 + 4 - NE])
        else:
            @pl.when(i + 7 < STEPS)
            def _():
                pltpu.async_copy(eidx_hbm.at[wid, i + 7], ev[(k8 + 7) % NE], a[(k8 + 7) % NE])

    for i in range(HEAD):
        _step(i, i % NB, i % NE, static=True)

    def octet(j, carry):
        for k in range(UNROLL):
            i = UNROLL * j + k
            _step(i, k % NB, k % NE, static=False)
        return carry

    lax.fori_loop(LOOP_LO, LOOP_HI, octet, 0)

    for i in range(TAIL_LO, STEPS):
        _step(i, i % NB, i % NE, static=True)


    plsc.subcore_barrier()
    pltpu.sync_copy(shared.at[pl.ds(r0, ROWS_A)],
                    out_hbm.at[c].at[pl.ds(r0, ROWS_A)])

    @pl.when(s == NS - 1)
    def _tail():
        t0 = NS * ROWS_A
        pltpu.sync_copy(shared.at[pl.ds(t0, ROWS_TAIL)],
                        out_hbm.at[c].at[pl.ds(t0, ROWS_TAIL)])


@functools.cache
def _make_agg():
    return pl.kernel(
        _agg_body,
        out_type=jax.ShapeDtypeStruct((NC, N, DIM), jnp.float32),
        mesh=plsc.VectorSubcoreMesh(core_axis_name="c", subcore_axis_name="s"),
        scratch_types=(
            [pltpu.VMEM_SHARED((N, DIM), jnp.float32)]
            + [pltpu.VMEM((CH, DIM), jnp.float32) for _ in range(NB)]
            + [pltpu.VMEM((2, CH), jnp.int32) for _ in range(NE)]
            + [pltpu.SemaphoreType.DMA for _ in range(2 * NB + NE)]
        ),
    )


def _layer_body(h_ref, agg_ref, batch_ref, w1_ref, b1_ref, w2_ref, b2_ref,
                gm_ref, bt_ref, hout_ref, pool_ref):
    m = h_ref[...] + agg_ref[0] + agg_ref[1]
    t = jnp.dot(m, w1_ref[...], preferred_element_type=jnp.float32) + b1_ref[...]
    t = jnp.maximum(t, 0.0)
    t = jnp.dot(t, w2_ref[...], preferred_element_type=jnp.float32) + b2_ref[...]
    t = jnp.maximum(t, 0.0)
    mu = jnp.mean(t, axis=0, keepdims=True)
    d = t - mu
    var = jnp.mean(d * d, axis=0, keepdims=True)
    hn = d * lax.rsqrt(var + 1e-5) * gm_ref[...] + bt_ref[...]
    hout_ref[...] = hn
    gids = lax.broadcasted_iota(jnp.int32, (G, N), 0)
    onehot = (batch_ref[...] == gids).astype(jnp.float32)
    pool_ref[...] = jnp.dot(onehot, hn, preferred_element_type=jnp.float32)


_layer = pl.pallas_call(
    _layer_body,
    out_shape=[
        jax.ShapeDtypeStruct((N, DIM), jnp.float32),
        jax.ShapeDtypeStruct((G, DIM), jnp.float32),
    ],
)


def _proj_body(p0_ref, p1_ref, p2_ref, P1_ref, pb1_ref, P2_ref, pb2_ref,
               cat_ref, proj_ref):
    cat = jnp.concatenate([p0_ref[...], p1_ref[...], p2_ref[...]], axis=1)
    cat_ref[...] = cat
    u = jnp.dot(cat, P1_ref[...], preferred_element_type=jnp.float32) + pb1_ref[...]
    u = jnp.maximum(u, 0.0)
    proj_ref[...] = jnp.dot(u, P2_ref[...], preferred_element_type=jnp.float32) + pb2_ref[...]


_proj = pl.pallas_call(
    _proj_body,
    out_shape=[
        jax.ShapeDtypeStruct((G, DIM * L), jnp.float32),
        jax.ShapeDtypeStruct((G, DIM * L), jnp.float32),
    ],
)


def kernel(x, edge_index, batch, mark, params):
    eidx = jnp.transpose(edge_index.reshape(2, NC * NS, STEPS, CH),
                         (1, 2, 0, 3))
    zeros = jnp.zeros((N, DIM), jnp.float32)
    batch2 = batch.reshape(1, N)
    h = x
    pooled = []
    agg_fn = _make_agg()
    for i in range(L):
        agg = agg_fn(h, eidx, zeros)
        h, p = _layer(
            h, agg, batch2,
            params[f"W1_{i}"], params[f"b1_{i}"].reshape(1, DIM),
            params[f"W2_{i}"], params[f"b2_{i}"].reshape(1, DIM),
            params[f"gamma_{i}"].reshape(1, DIM), params[f"beta_{i}"].reshape(1, DIM),
        )
        pooled.append(p)
    cat, proj = _proj(
        pooled[0], pooled[1], pooled[2],
        params["P1"], params["pb1"].reshape(1, DIM * L),
        params["P2"], params["pb2"].reshape(1, DIM * L),
    )
    return jnp.where(mark == 1, proj, cat)


# speedup vs baseline: 1.1778x; 1.1778x over previous
import functools

import jax
import jax.numpy as jnp
from jax import lax
from jax.experimental import pallas as pl
from jax.experimental.pallas import tpu as pltpu
from jax.experimental.pallas import tpu_sc as plsc

N = 10000
E = 320000
DIM = 128
G = 128
L = 3

NC = 2
NS = 16
CH = 80
E_TILE = E // (NC * NS)
STEPS = E_TILE // CH
NB = 4
NE = 8
ROWS_A = 624
ROWS_TAIL = N - NS * ROWS_A

UNROLL = 8
HEAD = 8
LOOP_LO = 1
LOOP_HI = STEPS // UNROLL
TAIL_LO = UNROLL * LOOP_HI


def _agg_body(h_hbm, eidx_hbm, zero_hbm, out_hbm, shared, *bufs):
    rows = bufs[0:NB]
    ev = bufs[NB:NB + NE]
    g = bufs[NB + NE:2 * NB + NE]
    sc = bufs[2 * NB + NE:3 * NB + NE]
    a = bufs[3 * NB + NE:3 * NB + 2 * NE]

    c = lax.axis_index("c")
    s = lax.axis_index("s")
    wid = c * NS + s

    r0 = pl.multiple_of(s * ROWS_A, 8)

    @pl.when(c == 0)
    def _init_h():
        pltpu.sync_copy(h_hbm.at[pl.ds(r0, ROWS_A)], shared.at[pl.ds(r0, ROWS_A)])

    @pl.when(c != 0)
    def _init_z():
        pltpu.sync_copy(zero_hbm.at[pl.ds(r0, ROWS_A)], shared.at[pl.ds(r0, ROWS_A)])

    @pl.when((s == NS - 1) & (c == 0))
    def _zth():
        t0 = NS * ROWS_A
        pltpu.sync_copy(h_hbm.at[pl.ds(t0, ROWS_TAIL)],
                        shared.at[pl.ds(t0, ROWS_TAIL)])

    @pl.when((s == NS - 1) & (c != 0))
    def _ztz():
        t0 = NS * ROWS_A
        pltpu.sync_copy(zero_hbm.at[pl.ds(t0, ROWS_TAIL)],
                        shared.at[pl.ds(t0, ROWS_TAIL)])

    def _wait_rows(buf, sem):
        pltpu.make_async_copy(h_hbm.at[pl.ds(0, CH)], buf, sem).wait()

    def _wait_idx(buf, sem):
        pltpu.make_async_copy(eidx_hbm.at[wid, 0], buf, sem).wait()

    def _wait_scat(sem):
        pltpu.make_async_copy(h_hbm.at[pl.ds(0, CH)], rows[0], sem).wait()

    for k in range(3):
        pltpu.sync_copy(eidx_hbm.at[wid, k], ev[k])
    plsc.subcore_barrier()
    for k in range(3):
        pltpu.async_copy(h_hbm.at[ev[k].at[0]], rows[k], g[k])
    for k in range(3, 7):
        pltpu.async_copy(eidx_hbm.at[wid, k], ev[k], a[k])

    def _step(i, k4, k8, static):
        kn4 = (k4 + 3) % NB
        kn8 = (k8 + 3) % NE
        _wait_rows(rows[k4], g[k4])

        def _gn():
            _wait_idx(ev[kn8], a[kn8])
            pltpu.async_copy(h_hbm.at[ev[kn8].at[0]], rows[kn4], g[kn4])

        if static:
            if i + 3 < STEPS:
                if i >= 1:
                    _wait_scat(sc[kn4])
                _gn()
        else:
            @pl.when(i + 3 < STEPS)
            def _():
                _wait_scat(sc[kn4])
                _gn()

        pltpu.async_copy(rows[k4], shared.at[ev[k8].at[1]], sc[k4], add=True)

        if static:
            if i + 7 < STEPS:
                pltpu.async_copy(eidx_hbm.at[wid, i + 7], ev[(k8 + 7) % NE], a[(k8 + 7) % NE])
        else:
            @pl.when(i + 7 < STEPS)
            def _():
                pltpu.async_copy(eidx_hbm.at[wid, i + 7], ev[(k8 + 7) % NE], a[(k8 + 7) % NE])

    for i in range(HEAD):
        _step(i, i % NB, i % NE, static=True)

    def octet(j, carry):
        for k in range(UNROLL):
            i = UNROLL * j + k
            _step(i, k % NB, k % NE, static=False)
        return carry

    lax.fori_loop(LOOP_LO, LOOP_HI, octet, 0)

    for i in range(TAIL_LO, STEPS):
        _step(i, i % NB, i % NE, static=True)

    for k in range(NB):
        _wait_scat(sc[k])

    plsc.subcore_barrier()
    pltpu.sync_copy(shared.at[pl.ds(r0, ROWS_A)],
                    out_hbm.at[c].at[pl.ds(r0, ROWS_A)])

    @pl.when(s == NS - 1)
    def _tail():
        t0 = NS * ROWS_A
        pltpu.sync_copy(shared.at[pl.ds(t0, ROWS_TAIL)],
                        out_hbm.at[c].at[pl.ds(t0, ROWS_TAIL)])


@functools.cache
def _make_agg():
    return pl.kernel(
        _agg_body,
        out_type=jax.ShapeDtypeStruct((NC, N, DIM), jnp.float32),
        mesh=plsc.VectorSubcoreMesh(core_axis_name="c", subcore_axis_name="s"),
        scratch_types=(
            [pltpu.VMEM_SHARED((N, DIM), jnp.float32)]
            + [pltpu.VMEM((CH, DIM), jnp.float32) for _ in range(NB)]
            + [pltpu.VMEM((2, CH), jnp.int32) for _ in range(NE)]
            + [pltpu.SemaphoreType.DMA for _ in range(2 * NB + NE)]
        ),
    )


def _layer_body(agg_ref, w1_ref, b1_ref, w2_ref, b2_ref,
                gm_ref, bt_ref, hout_ref):
    m = agg_ref[0] + agg_ref[1]
    t = jnp.dot(m, w1_ref[...], preferred_element_type=jnp.float32) + b1_ref[...]
    t = jnp.maximum(t, 0.0)
    t = jnp.dot(t, w2_ref[...], preferred_element_type=jnp.float32) + b2_ref[...]
    t = jnp.maximum(t, 0.0)
    mu = jnp.mean(t, axis=0, keepdims=True)
    d = t - mu
    var = jnp.mean(d * d, axis=0, keepdims=True)
    hout_ref[...] = d * lax.rsqrt(var + 1e-5) * gm_ref[...] + bt_ref[...]


_layer = pl.pallas_call(
    _layer_body,
    out_shape=jax.ShapeDtypeStruct((N, DIM), jnp.float32),
)


def _pool_body(h_ref, batch_ref, pool_ref):
    gids = lax.broadcasted_iota(jnp.int32, (G, N), 0)
    onehot = (batch_ref[...] == gids).astype(jnp.float32)
    pool_ref[...] = jnp.dot(onehot, h_ref[...], preferred_element_type=jnp.float32)


_pool = pl.pallas_call(
    _pool_body,
    out_shape=jax.ShapeDtypeStruct((G, DIM), jnp.float32),
)


def _proj_body(p0_ref, p1_ref, p2_ref, P1_ref, pb1_ref, P2_ref, pb2_ref,
               cat_ref, proj_ref):
    cat = jnp.concatenate([p0_ref[...], p1_ref[...], p2_ref[...]], axis=1)
    cat_ref[...] = cat
    u = jnp.dot(cat, P1_ref[...], preferred_element_type=jnp.float32) + pb1_ref[...]
    u = jnp.maximum(u, 0.0)
    proj_ref[...] = jnp.dot(u, P2_ref[...], preferred_element_type=jnp.float32) + pb2_ref[...]


_proj = pl.pallas_call(
    _proj_body,
    out_shape=[
        jax.ShapeDtypeStruct((G, DIM * L), jnp.float32),
        jax.ShapeDtypeStruct((G, DIM * L), jnp.float32),
    ],
)


def kernel(x, edge_index, batch, mark, params):
    eidx = jnp.transpose(edge_index.reshape(2, NC * NS, STEPS, CH),
                         (1, 2, 0, 3))
    zeros = jnp.zeros((N, DIM), jnp.float32)
    batch2 = batch.reshape(1, N)
    h = x
    pooled = []
    agg_fn = _make_agg()
    for i in range(L):
        agg = agg_fn(h, eidx, zeros)
        h = _layer(
            agg,
            params[f"W1_{i}"], params[f"b1_{i}"].reshape(1, DIM),
            params[f"W2_{i}"], params[f"b2_{i}"].reshape(1, DIM),
            params[f"gamma_{i}"].reshape(1, DIM), params[f"beta_{i}"].reshape(1, DIM),
        )
        pooled.append(_pool(h, batch2))
    cat, proj = _proj(
        pooled[0], pooled[1], pooled[2],
        params["P1"], params["pb1"].reshape(1, DIM * L),
        params["P2"], params["pb2"].reshape(1, DIM * L),
    )
    return jnp.where(mark == 1, proj, cat)
